# baseline (speedup 1.0000x reference)
import jax
import jax.numpy as jnp
from jax.experimental import pallas as pl

SEQ = 50
SS = 100
CLUSTER = 10
EMBED = 64
HID = 128
BB = 64


def _rep_block(cl, emb3, tgt, wlT, wrT, wtl, wtr, b):
    rep = jnp.zeros(tgt.shape, jnp.float32)
    for c in range(1, CLUSTER + 1):
        m = (cl == c).astype(jnp.float32)
        ec = jnp.sum(emb3 * m[:, :, None], axis=1)
        tl = jnp.dot(jnp.abs(ec - tgt), wlT, preferred_element_type=jnp.float32)
        tr = jnp.dot(ec * tgt, wrT, preferred_element_type=jnp.float32)
        a = jnp.tanh(jnp.dot(tl, wtl, preferred_element_type=jnp.float32)
                     + jnp.dot(tr, wtr, preferred_element_type=jnp.float32) + b)
        rep = rep + ec * a
    return rep


def _feat_kernel(ue_ref, ie_ref, ubc_ref, ihc_ref, ul3c_ref, il3c_ref,
                 ul1e_ref, il1e_ref, ul2e_ref, il2e_ref,
                 u1wl_ref, u1wr_ref, u1tl_ref, u1tr_ref, u1b_ref,
                 i1wl_ref, i1wr_ref, i1tl_ref, i1tr_ref, i1b_ref,
                 u2wl_ref, u2wr_ref, u2tl_ref, u2tr_ref, u2b_ref,
                 i2wl_ref, i2wr_ref, i2tl_ref, i2tr_ref, i2b_ref,
                 cat_ref):
    ue = ue_ref[...]
    ie = ie_ref[...]
    u_l1_rep = _rep_block(ubc_ref[...], ul1e_ref[...], ie,
                          u1wl_ref[...], u1wr_ref[...], u1tl_ref[...],
                          u1tr_ref[...], u1b_ref[...])
    i_l1_rep = _rep_block(ihc_ref[...], il1e_ref[...], ue,
                          i1wl_ref[...], i1wr_ref[...], i1tl_ref[...],
                          i1tr_ref[...], i1b_ref[...])
    u_l2_rep = _rep_block(ul3c_ref[...], ul2e_ref[...], ue,
                          u2wl_ref[...], u2wr_ref[...], u2tl_ref[...],
                          u2tr_ref[...], u2b_ref[...])
    i_l2_rep = _rep_block(il3c_ref[...], il2e_ref[...], ie,
                          i2wl_ref[...], i2wr_ref[...], i2tl_ref[...],
                          i2tr_ref[...], i2b_ref[...])
    cat_ref[...] = jnp.concatenate(
        [ue, ie, u_l1_rep, i_l1_rep, u_l2_rep, i_l2_rep,
         ue * ie, ue * i_l1_rep, ue * i_l2_rep,
         u_l1_rep * ie, u_l1_rep * i_l1_rep, u_l1_rep * i_l2_rep,
         u_l2_rep * ie, u_l2_rep * i_l1_rep, u_l2_rep * i_l2_rep], axis=1)


def _dice(x, alpha):
    mean = jnp.mean(x, axis=0, keepdims=True)
    var = jnp.mean(x * x, axis=0, keepdims=True) - mean * mean
    xn = (x - mean) * jax.lax.rsqrt(var + 1e-8)
    p = jax.nn.sigmoid(xn)
    return p * x + (1.0 - p) * alpha * x


def _mlp_kernel(cat_ref, w1_ref, b1_ref, a1_ref, w2_ref, b2_ref, a2_ref,
                w3_ref, b3_ref, out_ref):
    h = jnp.dot(cat_ref[...], w1_ref[...],
                preferred_element_type=jnp.float32) + b1_ref[...]
    h = _dice(h, a1_ref[...])
    h = jnp.dot(h, w2_ref[...], preferred_element_type=jnp.float32) + b2_ref[...]
    h = _dice(h, a2_ref[...])
    o = jnp.dot(h, w3_ref[...], preferred_element_type=jnp.float32) + b3_ref[...]
    out_ref[...] = jax.nn.sigmoid(o)


def kernel(x, cate_list, u_cluster_list, i_cluster_list, W_user, W_item, W_cate,
           ul1_wl, ul1_wr, ul1_wt_w, ul1_wt_b,
           il1_wl, il1_wr, il1_wt_w, il1_wt_b,
           ul2_wl, ul2_wr, ul2_wt_w, ul2_wt_b,
           il2_wl, il2_wr, il2_wt_w, il2_wt_b,
           mlp_w1, mlp_b1, alpha1, mlp_w2, mlp_b2, alpha2, mlp_w3, mlp_b3):
    B = x.shape[0]
    S, T = SEQ, SS
    uid = x[:, 0]
    uba = x[:, 1:S + 1]
    ul3u = x[:, S + 1:S + T + 1]
    tgt = x[:, S + T + 1]
    ihu = x[:, S + T + 2:2 * S + T + 2]
    il3i = x[:, 2 * S + T + 2:]

    user_emb = W_user[uid]
    item_emb = W_item[tgt] + W_cate[cate_list[tgt]]
    ubc = i_cluster_list[uba]
    ul3c = u_cluster_list[ul3u]
    ihc = u_cluster_list[ihu]
    il3c = i_cluster_list[il3i]
    u_l1 = W_item[uba] + W_cate[cate_list[uba]]
    u_l2 = W_user[ul3u]
    i_l1 = W_user[ihu]
    i_l2 = W_item[il3i] + W_cate[cate_list[il3i]]

    def attw(wl, wr, wt_w, wt_b):
        return (wl.T, wr.T, wt_w[:, :HID].T, wt_w[:, HID:].T,
                wt_b.reshape(1, 1))

    aw = (attw(ul1_wl, ul1_wr, ul1_wt_w, ul1_wt_b)
          + attw(il1_wl, il1_wr, il1_wt_w, il1_wt_b)
          + attw(ul2_wl, ul2_wr, ul2_wt_w, ul2_wt_b)
          + attw(il2_wl, il2_wr, il2_wt_w, il2_wt_b))

    nb = B // BB
    row2 = lambda i: (i, 0)
    row3 = lambda i: (i, 0, 0)
    full2 = lambda i: (0, 0)
    wspecs = []
    for w in aw:
        wspecs.append(pl.BlockSpec(w.shape, full2))

    cat = pl.pallas_call(
        _feat_kernel,
        grid=(nb,),
        in_specs=[
            pl.BlockSpec((BB, EMBED), row2),
            pl.BlockSpec((BB, EMBED), row2),
            pl.BlockSpec((BB, S), row2),
            pl.BlockSpec((BB, S), row2),
            pl.BlockSpec((BB, T), row2),
            pl.BlockSpec((BB, T), row2),
            pl.BlockSpec((BB, S, EMBED), row3),
            pl.BlockSpec((BB, S, EMBED), row3),
            pl.BlockSpec((BB, T, EMBED), row3),
            pl.BlockSpec((BB, T, EMBED), row3),
        ] + wspecs,
        out_specs=pl.BlockSpec((BB, 15 * EMBED), row2),
        out_shape=jax.ShapeDtypeStruct((B, 15 * EMBED), jnp.float32),
    )(user_emb, item_emb, ubc, ihc, ul3c, il3c, u_l1, i_l1, u_l2, i_l2, *aw)

    mw = (mlp_w1.T, mlp_b1.reshape(1, -1), alpha1.reshape(1, -1),
          mlp_w2.T, mlp_b2.reshape(1, -1), alpha2.reshape(1, -1),
          mlp_w3.T, mlp_b3.reshape(1, 1))

    out = pl.pallas_call(
        _mlp_kernel,
        out_shape=jax.ShapeDtypeStruct((B, 1), jnp.float32),
    )(cat, *mw)
    return out


# speedup vs baseline: 1.6205x; 1.6205x over previous
import jax
import jax.numpy as jnp
from jax.experimental import pallas as pl

SEQ = 50
SS = 100
CLUSTER = 10
EMBED = 64
HID = 128
BB = 128


def _rep_block(cl, emb3, tgt, wlT, wrT, wtl, wtr, b):
    n, s = cl.shape
    cidx = jax.lax.broadcasted_iota(jnp.int32, (n, CLUSTER, s), 1) + 1
    mask = (cl[:, None, :] == cidx).astype(jnp.float32)
    embs = jax.lax.dot_general(mask, emb3, (((2,), (1,)), ((0,), (0,))),
                               preferred_element_type=jnp.float32)
    r = embs.reshape(n * CLUSTER, EMBED)
    tb = jnp.broadcast_to(tgt[:, None, :], (n, CLUSTER, EMBED))
    tb = tb.reshape(n * CLUSTER, EMBED)
    tl = jnp.dot(jnp.abs(r - tb), wlT, preferred_element_type=jnp.float32)
    tr = jnp.dot(r * tb, wrT, preferred_element_type=jnp.float32)
    a = jnp.tanh(jnp.dot(tl, wtl, preferred_element_type=jnp.float32)
                 + jnp.dot(tr, wtr, preferred_element_type=jnp.float32) + b)
    return jnp.sum((r * a).reshape(n, CLUSTER, EMBED), axis=1)


def _feat_kernel(ue_ref, ie_ref, ubc_ref, ihc_ref, ul3c_ref, il3c_ref,
                 ul1e_ref, il1e_ref, ul2e_ref, il2e_ref,
                 u1wl_ref, u1wr_ref, u1tl_ref, u1tr_ref, u1b_ref,
                 i1wl_ref, i1wr_ref, i1tl_ref, i1tr_ref, i1b_ref,
                 u2wl_ref, u2wr_ref, u2tl_ref, u2tr_ref, u2b_ref,
                 i2wl_ref, i2wr_ref, i2tl_ref, i2tr_ref, i2b_ref,
                 w1_ref, b1_ref, h1_ref):
    ue = ue_ref[...]
    ie = ie_ref[...]
    u_l1_rep = _rep_block(ubc_ref[...], ul1e_ref[...], ie,
                          u1wl_ref[...], u1wr_ref[...], u1tl_ref[...],
                          u1tr_ref[...], u1b_ref[...])
    i_l1_rep = _rep_block(ihc_ref[...], il1e_ref[...], ue,
                          i1wl_ref[...], i1wr_ref[...], i1tl_ref[...],
                          i1tr_ref[...], i1b_ref[...])
    u_l2_rep = _rep_block(ul3c_ref[...], ul2e_ref[...], ue,
                          u2wl_ref[...], u2wr_ref[...], u2tl_ref[...],
                          u2tr_ref[...], u2b_ref[...])
    i_l2_rep = _rep_block(il3c_ref[...], il2e_ref[...], ie,
                          i2wl_ref[...], i2wr_ref[...], i2tl_ref[...],
                          i2tr_ref[...], i2b_ref[...])
    cat = jnp.concatenate(
        [ue, ie, u_l1_rep, i_l1_rep, u_l2_rep, i_l2_rep,
         ue * ie, ue * i_l1_rep, ue * i_l2_rep,
         u_l1_rep * ie, u_l1_rep * i_l1_rep, u_l1_rep * i_l2_rep,
         u_l2_rep * ie, u_l2_rep * i_l1_rep, u_l2_rep * i_l2_rep], axis=1)
    h1_ref[...] = jnp.dot(cat, w1_ref[...],
                          preferred_element_type=jnp.float32) + b1_ref[...]


def _dice(x, alpha):
    mean = jnp.mean(x, axis=0, keepdims=True)
    var = jnp.mean(x * x, axis=0, keepdims=True) - mean * mean
    xn = (x - mean) * jax.lax.rsqrt(var + 1e-8)
    p = jax.nn.sigmoid(xn)
    return p * x + (1.0 - p) * alpha * x


def _mlp_kernel(h1_ref, a1_ref, w2_ref, b2_ref, a2_ref,
                w3_ref, b3_ref, out_ref):
    h = _dice(h1_ref[...], a1_ref[...])
    h = jnp.dot(h, w2_ref[...], preferred_element_type=jnp.float32) + b2_ref[...]
    h = _dice(h, a2_ref[...])
    o = jnp.dot(h, w3_ref[...], preferred_element_type=jnp.float32) + b3_ref[...]
    out_ref[...] = jax.nn.sigmoid(o)


def kernel(x, cate_list, u_cluster_list, i_cluster_list, W_user, W_item, W_cate,
           ul1_wl, ul1_wr, ul1_wt_w, ul1_wt_b,
           il1_wl, il1_wr, il1_wt_w, il1_wt_b,
           ul2_wl, ul2_wr, ul2_wt_w, ul2_wt_b,
           il2_wl, il2_wr, il2_wt_w, il2_wt_b,
           mlp_w1, mlp_b1, alpha1, mlp_w2, mlp_b2, alpha2, mlp_w3, mlp_b3):
    B = x.shape[0]
    S, T = SEQ, SS
    uid = x[:, 0]
    uba = x[:, 1:S + 1]
    ul3u = x[:, S + 1:S + T + 1]
    tgt = x[:, S + T + 1]
    ihu = x[:, S + T + 2:2 * S + T + 2]
    il3i = x[:, 2 * S + T + 2:]

    user_emb = W_user[uid]
    item_emb = W_item[tgt] + W_cate[cate_list[tgt]]
    ubc = i_cluster_list[uba]
    ul3c = u_cluster_list[ul3u]
    ihc = u_cluster_list[ihu]
    il3c = i_cluster_list[il3i]
    u_l1 = W_item[uba] + W_cate[cate_list[uba]]
    u_l2 = W_user[ul3u]
    i_l1 = W_user[ihu]
    i_l2 = W_item[il3i] + W_cate[cate_list[il3i]]

    def attw(wl, wr, wt_w, wt_b):
        return (wl.T, wr.T, wt_w[:, :HID].T, wt_w[:, HID:].T,
                wt_b.reshape(1, 1))

    aw = (attw(ul1_wl, ul1_wr, ul1_wt_w, ul1_wt_b)
          + attw(il1_wl, il1_wr, il1_wt_w, il1_wt_b)
          + attw(ul2_wl, ul2_wr, ul2_wt_w, ul2_wt_b)
          + attw(il2_wl, il2_wr, il2_wt_w, il2_wt_b)
          + (mlp_w1.T, mlp_b1.reshape(1, -1)))

    nb = B // BB
    row2 = lambda i: (i, 0)
    row3 = lambda i: (i, 0, 0)
    full2 = lambda i: (0, 0)
    wspecs = [pl.BlockSpec(w.shape, full2) for w in aw]

    h1 = pl.pallas_call(
        _feat_kernel,
        grid=(nb,),
        in_specs=[
            pl.BlockSpec((BB, EMBED), row2),
            pl.BlockSpec((BB, EMBED), row2),
            pl.BlockSpec((BB, S), row2),
            pl.BlockSpec((BB, S), row2),
            pl.BlockSpec((BB, T), row2),
            pl.BlockSpec((BB, T), row2),
            pl.BlockSpec((BB, S, EMBED), row3),
            pl.BlockSpec((BB, S, EMBED), row3),
            pl.BlockSpec((BB, T, EMBED), row3),
            pl.BlockSpec((BB, T, EMBED), row3),
        ] + wspecs,
        out_specs=pl.BlockSpec((BB, 200), row2),
        out_shape=jax.ShapeDtypeStruct((B, 200), jnp.float32),
    )(user_emb, item_emb, ubc, ihc, ul3c, il3c, u_l1, i_l1, u_l2, i_l2, *aw)

    mw = (alpha1.reshape(1, -1), mlp_w2.T, mlp_b2.reshape(1, -1),
          alpha2.reshape(1, -1), mlp_w3.T, mlp_b3.reshape(1, 1))

    out = pl.pallas_call(
        _mlp_kernel,
        out_shape=jax.ShapeDtypeStruct((B, 1), jnp.float32),
    )(h1, *mw)
    return out
